# baseline (speedup 1.0000x reference)
import jax
import jax.numpy as jnp
from jax.experimental import pallas as pl

_BL = 512
_SB = 16


def _body(tok_ref, en_ref, w_ref, eb_ref, pe_ref, out_ref):
    e = jax.lax.dot_general(
        w_ref[0], en_ref[:], (((0,), (0,)), ((), ())),
        preferred_element_type=jnp.float32)
    out_ref[0, :, :] = e + eb_ref[:]
    out_ref[1:, :, :] = tok_ref[:] + pe_ref[:]


def kernel(tokens, energies, W, b, emb):
    batch, n_in, tsz = tokens.shape
    n_tok = emb.shape[0]
    tokens_t = tokens.transpose(1, 2, 0)
    energies_t = energies.T
    pe = emb[1:].reshape(n_in, tsz, 1)
    eb = (b + emb[0]).reshape(tsz, 1)
    w_r = W.reshape(tsz, tsz // _SB, _SB).transpose(1, 0, 2)

    grid = (batch // _BL, tsz // _SB)
    out_t = pl.pallas_call(
        _body,
        grid=grid,
        in_specs=[
            pl.BlockSpec((n_in, _SB, _BL), lambda j, k: (0, k, j)),
            pl.BlockSpec((tsz, _BL), lambda j, k: (0, j)),
            pl.BlockSpec((1, tsz, _SB), lambda j, k: (k, 0, 0)),
            pl.BlockSpec((_SB, 1), lambda j, k: (k, 0)),
            pl.BlockSpec((n_in, _SB, 1), lambda j, k: (0, k, 0)),
        ],
        out_specs=pl.BlockSpec((n_tok, _SB, _BL), lambda j, k: (0, k, j)),
        out_shape=jax.ShapeDtypeStruct((n_tok, tsz, batch), jnp.float32),
    )(tokens_t, energies_t, w_r, eb, pe)
    return out_t.transpose(2, 0, 1)


# speedup vs baseline: 1.1158x; 1.1158x over previous
import jax
import jax.numpy as jnp
from jax.experimental import pallas as pl
from jax.experimental.pallas import tpu as pltpu

_BL = 256


def _body(tok_ref, en_ref, w_ref, eb_ref, pe_ref, out_ref):
    j = pl.program_id(0)
    e = jax.lax.dot_general(
        w_ref[:], en_ref[:, pl.ds(j * _BL, _BL)], (((0,), (0,)), ((), ())),
        preferred_element_type=jnp.float32)
    out_ref[0, :, :] = e + eb_ref[:]
    out_ref[1:, :, :] = tok_ref[:] + pe_ref[:]


def kernel(tokens, energies, W, b, emb):
    batch, n_in, tsz = tokens.shape
    n_tok = emb.shape[0]
    tokens_t = tokens.transpose(1, 2, 0)
    energies_t = energies.T
    pe = emb[1:].reshape(n_in, tsz, 1)
    eb = (b + emb[0]).reshape(tsz, 1)

    grid = (batch // _BL,)
    resident = pl.BlockSpec(memory_space=pltpu.VMEM)
    out_t = pl.pallas_call(
        _body,
        grid=grid,
        in_specs=[
            pl.BlockSpec((n_in, tsz, _BL), lambda j: (0, 0, j)),
            resident,
            resident,
            resident,
            resident,
        ],
        out_specs=pl.BlockSpec((n_tok, tsz, _BL), lambda j: (0, 0, j)),
        out_shape=jax.ShapeDtypeStruct((n_tok, tsz, batch), jnp.float32),
    )(tokens_t, energies_t, W, eb, pe)
    return out_t.transpose(2, 0, 1)
